# Initial kernel scaffold; baseline (speedup 1.0000x reference)
import jax
import jax.numpy as jnp
from jax.experimental import pallas as pl


def kernel(all_rois, gt_boxes, gt_masks, mask_gt_boxes, ratios):
    raise NotImplementedError("write your pallas kernel here")


# speedup vs baseline: 2.0637x; 2.0637x over previous
import functools

import jax
import jax.numpy as jnp
from jax import lax
from jax.experimental import pallas as pl
from jax.experimental.pallas import tpu as pltpu

FG_THRESH = 0.7
BG_THRESH_HI = 0.3
BG_THRESH_LO = 0.0
ROIS = 256
FG = 64
NEG = -1e9
PAD_SCORE = -2e9
DONE_SCORE = -3e9


def _body(nt, rois_ln_ref, rois_nt_ref, gt_smem, gt_v, mgt_v, masks_ref,
          ratios_smem, o_rois, o_small, o_w, o_masks, sel_ref):
    b = pl.program_id(0)
    nr = rois_ln_ref.shape[2]
    g = gt_v.shape[2]

    x1 = rois_ln_ref[0, 0]
    y1 = rois_ln_ref[0, 1]
    x2 = rois_ln_ref[0, 2]
    y2 = rois_ln_ref[0, 3]
    area_a = (x2 - x1 + 1.0) * (y2 - y1 + 1.0)

    def gbody(gi, mx):
        gx1 = gt_smem[b, 1, gi]
        gy1 = gt_smem[b, 2, gi]
        gx2 = gt_smem[b, 3, gi]
        gy2 = gt_smem[b, 4, gi]
        ix1 = jnp.maximum(x1, gx1)
        iy1 = jnp.maximum(y1, gy1)
        ix2 = jnp.minimum(x2, gx2)
        iy2 = jnp.minimum(y2, gy2)
        iw = jnp.maximum(ix2 - ix1 + 1.0, 0.0)
        ih = jnp.maximum(iy2 - iy1 + 1.0, 0.0)
        inter = iw * ih
        area_b = (gx2 - gx1 + 1.0) * (gy2 - gy1 + 1.0)
        iou = inter / (area_a + area_b - inter + 1e-6)
        return jnp.maximum(mx, iou)

    mx = lax.fori_loop(0, g, gbody, jnp.full((nr, 128), -jnp.inf, jnp.float32))

    ridx = (lax.broadcasted_iota(jnp.int32, (nr, 128), 0) * 128
            + lax.broadcasted_iota(jnp.int32, (nr, 128), 1))
    valid = ridx < nt
    pad_fill = jnp.where(valid, NEG, PAD_SCORE).astype(jnp.float32)
    fgs = jnp.where(valid & (mx >= FG_THRESH), mx, pad_fill)
    bgs = jnp.where(valid & (mx < BG_THRESH_HI) & (mx >= BG_THRESH_LO),
                    mx, pad_fill)

    def make_sel(offset):
        def body(t, s):
            m = jnp.max(s)
            idx = jnp.min(jnp.where(s == m, ridx, jnp.int32(2**30)))
            row = rois_nt_ref[0, pl.ds(idx, 1), :]
            sel_ref[pl.ds(offset + t, 1), :] = row
            return jnp.where(ridx == idx, DONE_SCORE, s)
        return body

    lax.fori_loop(0, FG, make_sel(0), fgs)
    lax.fori_loop(0, ROIS - FG, make_sel(FG), bgs)

    sel = sel_ref[...]
    ex1 = sel[:, 1:2]
    ey1 = sel[:, 2:3]
    ex2 = sel[:, 3:4]
    ey2 = sel[:, 4:5]
    earea = (ex2 - ex1 + 1.0) * (ey2 - ey1 + 1.0)
    liota = lax.broadcasted_iota(jnp.int32, (ROIS, g), 1)

    def iou_vs(gref):
        gx1 = gref[1:2, :]
        gy1 = gref[2:3, :]
        gx2 = gref[3:4, :]
        gy2 = gref[4:5, :]
        ix1 = jnp.maximum(ex1, gx1)
        iy1 = jnp.maximum(ey1, gy1)
        ix2 = jnp.minimum(ex2, gx2)
        iy2 = jnp.minimum(ey2, gy2)
        iw = jnp.maximum(ix2 - ix1 + 1.0, 0.0)
        ih = jnp.maximum(iy2 - iy1 + 1.0, 0.0)
        inter = iw * ih
        garea = (gx2 - gx1 + 1.0) * (gy2 - gy1 + 1.0)
        iou = inter / (earea + garea - inter + 1e-6)
        mo = jnp.max(iou, axis=1, keepdims=True)
        asg = jnp.min(jnp.where(iou == mo, liota, jnp.int32(g)),
                      axis=1, keepdims=True)
        onehot = (liota == asg).astype(jnp.float32)
        return mo, onehot

    gtv = gt_v[0]
    mo_g, oh_g = iou_vs(gtv)
    labels_keep = jnp.sum(oh_g * gtv[5:6, :], axis=1, keepdims=True)
    gx1s = jnp.sum(oh_g * gtv[1:2, :], axis=1, keepdims=True)
    gy1s = jnp.sum(oh_g * gtv[2:3, :], axis=1, keepdims=True)
    gx2s = jnp.sum(oh_g * gtv[3:4, :], axis=1, keepdims=True)
    gy2s = jnp.sum(oh_g * gtv[4:5, :], axis=1, keepdims=True)

    pos = lax.broadcasted_iota(jnp.int32, (ROIS, 1), 0)
    is_fg = (pos < FG) & (mo_g >= FG_THRESH)
    fgf = is_fg.astype(jnp.float32)
    labels_b = jnp.where(is_fg, labels_keep, 0.0)

    ew = ex2 - ex1 + 1.0
    eh = ey2 - ey1 + 1.0
    r0 = ratios_smem[b, 0]
    r1 = ratios_smem[b, 1]
    tlx = jnp.where(is_fg, (gx1s - ex1) / ew * r0, 0.0)
    tly = jnp.where(is_fg, (gy1s - ey1) / eh * r1, 0.0)
    brx = jnp.where(is_fg, (gx2s - ex2) / ew * r0, 0.0)
    bry = jnp.where(is_fg, (gy2s - ey2) / eh * r1, 0.0)

    mgtv = mgt_v[0]
    mo_m, oh_m = iou_vs(mgtv)
    msel = (mo_m >= FG_THRESH).astype(jnp.float32)
    mlab = jnp.sum(oh_m * mgtv[5:6, :], axis=1, keepdims=True) * msel

    o_rois[0] = sel
    o_small[0] = jnp.concatenate(
        [labels_b, fgf, msel, mlab, tlx, tly, brx, bry], axis=1)
    o_w[0] = jnp.broadcast_to(fgf, (ROIS, 4))
    o_masks[0] = jnp.dot(oh_m, masks_ref[0],
                         preferred_element_type=jnp.float32,
                         precision=lax.Precision.HIGHEST)


def kernel(all_rois, gt_boxes, gt_masks, mask_gt_boxes, ratios):
    b, n, _ = all_rois.shape
    g = gt_boxes.shape[1]
    nt = n + g
    npad = ((nt + 1023) // 1024) * 1024
    nr = npad // 128
    mhw = gt_masks.shape[2] * gt_masks.shape[3]

    rois_full = jnp.concatenate(
        [all_rois[:, :, :1], all_rois[:, :, 1:5] * 8.0, all_rois[:, :, 5:]],
        axis=2)
    rois_full = jnp.concatenate([rois_full, gt_boxes], axis=1)
    rois_nt = jnp.pad(rois_full, ((0, 0), (0, npad - nt), (0, 1)))
    coords = jnp.transpose(rois_full[:, :, 1:5], (0, 2, 1))
    coords = jnp.pad(coords, ((0, 0), (0, 0), (0, npad - nt)))
    rois_ln = coords.reshape(b, 4, nr, 128)
    gt_t = jnp.pad(jnp.transpose(gt_boxes, (0, 2, 1)), ((0, 0), (0, 1), (0, 0)))
    mgt_t = jnp.pad(jnp.transpose(mask_gt_boxes, (0, 2, 1)),
                    ((0, 0), (0, 1), (0, 0)))
    masks2 = gt_masks.reshape(b, g, mhw)

    out_shapes = (
        jax.ShapeDtypeStruct((b, ROIS, 8), jnp.float32),
        jax.ShapeDtypeStruct((b, ROIS, 8), jnp.float32),
        jax.ShapeDtypeStruct((b, ROIS, 4), jnp.float32),
        jax.ShapeDtypeStruct((b, ROIS, mhw), jnp.float32),
    )
    o_rois, o_small, o_w, o_masks = pl.pallas_call(
        functools.partial(_body, nt),
        grid=(b,),
        in_specs=[
            pl.BlockSpec((1, 4, nr, 128), lambda i: (i, 0, 0, 0)),
            pl.BlockSpec((1, npad, 8), lambda i: (i, 0, 0)),
            pl.BlockSpec(memory_space=pltpu.SMEM),
            pl.BlockSpec((1, 8, g), lambda i: (i, 0, 0)),
            pl.BlockSpec((1, 8, g), lambda i: (i, 0, 0)),
            pl.BlockSpec((1, g, mhw), lambda i: (i, 0, 0)),
            pl.BlockSpec(memory_space=pltpu.SMEM),
        ],
        out_specs=(
            pl.BlockSpec((1, ROIS, 8), lambda i: (i, 0, 0)),
            pl.BlockSpec((1, ROIS, 8), lambda i: (i, 0, 0)),
            pl.BlockSpec((1, ROIS, 4), lambda i: (i, 0, 0)),
            pl.BlockSpec((1, ROIS, mhw), lambda i: (i, 0, 0)),
        ),
        out_shape=out_shapes,
        scratch_shapes=[pltpu.VMEM((ROIS, 8), jnp.float32)],
    )(rois_ln, rois_nt, gt_t, gt_t, mgt_t, masks2, ratios)

    rois_batch = o_rois[:, :, :7]
    labels_batch = o_small[:, :, 0]
    bbox_tl = o_small[:, :, 4:6]
    bbox_br = o_small[:, :, 6:8]
    target_masks = o_masks.reshape(b, ROIS, gt_masks.shape[2],
                                   gt_masks.shape[3])
    mask_select = o_small[:, :, 2]
    mask_labels = o_small[:, :, 3]
    return (rois_batch, labels_batch, bbox_tl, bbox_br, o_w, o_w,
            target_masks, mask_select, mask_labels)
